# baseline (speedup 1.0000x reference)
import functools
import math

import jax
import jax.numpy as jnp
from jax import lax
from jax.experimental import pallas as pl
from jax.experimental.pallas import tpu as pltpu
from jax.experimental.pallas import tpu_sc as plsc

_NUM_SAMPLED = 5
_NW = 32
_CHUNK = 128


@functools.lru_cache(maxsize=4)
def _sample_negs(n, num_nodes):
    with jax.ensure_compile_time_eval():
        kk = jnp.arange(num_nodes, dtype=jnp.float32)
        sw = (jnp.log(kk + 2.0) - jnp.log(kk + 1.0)) / math.log(num_nodes + 1)
        sw = sw / jnp.linalg.norm(sw)
        negs = jax.random.choice(jax.random.key(12345), num_nodes,
                                 shape=(n, _NUM_SAMPLED), replace=True, p=sw)
        return jax.device_get(negs.T.reshape(-1).astype(jnp.int32))


def _sc_scores(wpacked, embs, idx3, d):
    nw, c_chunks, chunk = idx3.shape
    n, _ = embs.shape
    dh = d // 2
    bb = n // nw
    sub = bb // chunk
    groups = chunk // 16
    mesh = plsc.VectorSubcoreMesh(core_axis_name="c", subcore_axis_name="s")

    @functools.partial(
        pl.kernel,
        mesh=mesh,
        out_type=jax.ShapeDtypeStruct((nw, c_chunks * chunk), jnp.float32),
        compiler_params=pltpu.CompilerParams(use_tc_tiling_on_sc=False,
                                             needs_layout_passes=False),
        scratch_types=[
            pltpu.VMEM((c_chunks, chunk), jnp.int32),
            pltpu.VMEM((bb, d), jnp.float32),
            pltpu.VMEM((chunk, dh), jnp.int32),
            pltpu.VMEM((chunk, dh), jnp.int32),
            pltpu.VMEM((c_chunks * chunk,), jnp.float32),
            pltpu.SemaphoreType.DMA,
            pltpu.SemaphoreType.DMA,
        ],
    )
    def k(w_hbm, e_hbm, idx_hbm, out_hbm, idx_v, e_v, bufa, bufb, s_v,
          sema, semb):
        wid = lax.axis_index("s") * 2 + lax.axis_index("c")
        pltpu.sync_copy(idx_hbm.at[wid], idx_v)
        pltpu.sync_copy(e_hbm.at[pl.ds(wid * bb, bb)], e_v)
        lanes = lax.iota(jnp.int32, 16)

        mask_hi = jnp.full((16,), -65536, jnp.int32)

        def compute(c, buf):
            j = lax.rem(c, sub)
            for g in range(groups):
                wrow = g * 16 + lanes
                erow = j * chunk + g * 16 + lanes
                acc = jnp.zeros((16,), jnp.float32)
                for dd in range(dh):
                    col = jnp.full((16,), dd, jnp.int32)
                    wv = plsc.load_gather(buf, [wrow, col])
                    lo = plsc.bitcast(lax.shift_left(wv, 16), jnp.float32)
                    hi = plsc.bitcast(lax.bitwise_and(wv, mask_hi), jnp.float32)
                    e0 = plsc.load_gather(e_v, [erow, jnp.full((16,), 2 * dd, jnp.int32)])
                    e1 = plsc.load_gather(e_v, [erow, jnp.full((16,), 2 * dd + 1, jnp.int32)])
                    acc = acc + lo * e0 + hi * e1
                s_v[pl.ds(c * chunk + g * 16, 16)] = acc

        def fire(c, buf, sem):
            return pltpu.async_copy(w_hbm.at[idx_v.at[c]], buf, sem)

        def wait(buf, sem):
            pltpu.make_async_copy(w_hbm.at[idx_v.at[0]], buf, sem).wait()

        fire(0, bufa, sema)

        def body(p, carry):
            c0 = 2 * p
            fire(c0 + 1, bufb, semb)
            wait(bufa, sema)
            compute(c0, bufa)

            @pl.when(p < c_chunks // 2 - 1)
            def _():
                fire(c0 + 2, bufa, sema)

            wait(bufb, semb)
            compute(c0 + 1, bufb)
            return carry

        lax.fori_loop(0, c_chunks // 2, body, 0)
        pltpu.sync_copy(s_v, out_hbm.at[wid])

    return k(wpacked, embs, idx3)


def _tc_loss_sum(scores, pos_cols):
    def body(s_ref, o_ref):
        s = s_ref[...]
        col = lax.broadcasted_iota(jnp.int32, s.shape, 1)
        z = jnp.where(col < pos_cols, s, -s)
        o_ref[...] = jnp.sum(jnp.log(jax.nn.sigmoid(z))).reshape(1, 1)

    out = pl.pallas_call(
        body,
        out_shape=jax.ShapeDtypeStruct((1, 1), jnp.float32),
    )(scores)
    return out[0, 0]


def kernel(input, embs, label, weights):
    n, d = embs.shape
    num_nodes = weights.shape[0]
    negs_flat = _sample_negs(n, num_nodes)
    idx = jnp.concatenate([label.astype(jnp.int32), jnp.asarray(negs_flat)])
    bb = n // _NW
    idx3 = (idx.reshape(_NUM_SAMPLED + 1, _NW, bb // _CHUNK, _CHUNK)
            .transpose(1, 0, 2, 3)
            .reshape(_NW, -1, _CHUNK))
    wpacked = jax.lax.bitcast_convert_type(
        weights.astype(jnp.bfloat16).reshape(num_nodes, d // 2, 2), jnp.int32)
    scores = _sc_scores(wpacked, embs, idx3, d)
    total = _tc_loss_sum(scores, bb)
    return -total / n


# speedup vs baseline: 2.2479x; 2.2479x over previous
import functools
import math

import jax
import jax.numpy as jnp
from jax import lax
from jax.experimental import pallas as pl
from jax.experimental.pallas import tpu as pltpu
from jax.experimental.pallas import tpu_sc as plsc

_NUM_SAMPLED = 5
_NW = 32
_CHUNK = 128
_NBUF = 4


@functools.lru_cache(maxsize=4)
def _sample_negs(n, num_nodes):
    with jax.ensure_compile_time_eval():
        kk = jnp.arange(num_nodes, dtype=jnp.float32)
        sw = (jnp.log(kk + 2.0) - jnp.log(kk + 1.0)) / math.log(num_nodes + 1)
        sw = sw / jnp.linalg.norm(sw)
        negs = jax.random.choice(jax.random.key(12345), num_nodes,
                                 shape=(n, _NUM_SAMPLED), replace=True, p=sw)
        return jax.device_get(negs.T.reshape(-1).astype(jnp.int32))


def _sc_scores(weights, embs, idx3):
    nw, c_chunks, chunk = idx3.shape
    n, d = embs.shape
    bb = n // nw
    sub = bb // chunk
    groups = chunk // 16
    mesh = plsc.VectorSubcoreMesh(core_axis_name="c", subcore_axis_name="s")

    @functools.partial(
        pl.kernel,
        mesh=mesh,
        out_type=jax.ShapeDtypeStruct((nw, c_chunks * chunk), jnp.float32),
        compiler_params=pltpu.CompilerParams(use_tc_tiling_on_sc=False,
                                             needs_layout_passes=False),
        scratch_types=[
            pltpu.VMEM((c_chunks, chunk), jnp.int32),
            pltpu.VMEM((bb, d), jnp.float32),
            pltpu.VMEM((c_chunks * chunk,), jnp.float32),
        ] + [pltpu.VMEM((chunk, d), jnp.float32)] * _NBUF
          + [pltpu.SemaphoreType.DMA] * _NBUF,
    )
    def k(w_hbm, e_hbm, idx_hbm, out_hbm, idx_v, e_v, s_v, *bufsem):
        bufs = bufsem[:_NBUF]
        sems = bufsem[_NBUF:]
        wid = lax.axis_index("s") * 2 + lax.axis_index("c")
        pltpu.sync_copy(idx_hbm.at[wid], idx_v)
        pltpu.sync_copy(e_hbm.at[pl.ds(wid * bb, bb)], e_v)
        lanes = lax.iota(jnp.int32, 16)

        def compute(c, buf):
            j = lax.rem(c, sub)
            for g in range(groups):
                wrow = g * 16 + lanes
                erow = j * chunk + g * 16 + lanes
                acc = jnp.zeros((16,), jnp.float32)
                for dd in range(d):
                    col = jnp.full((16,), dd, jnp.int32)
                    wv = plsc.load_gather(buf, [wrow, col])
                    ev = plsc.load_gather(e_v, [erow, col])
                    acc = acc + wv * ev
                s_v[pl.ds(c * chunk + g * 16, 16)] = acc

        def fire(c, k_):
            pltpu.async_copy(w_hbm.at[idx_v.at[c]], bufs[k_], sems[k_])

        def wait(k_):
            pltpu.make_async_copy(w_hbm.at[idx_v.at[0]], bufs[k_],
                                  sems[k_]).wait()

        for k_ in range(_NBUF):
            fire(k_, k_)

        def body(p, carry):
            for k_ in range(_NBUF):
                c = _NBUF * p + k_
                wait(k_)
                compute(c, bufs[k_])

                @pl.when(c + _NBUF < c_chunks)
                def _():
                    fire(c + _NBUF, k_)
            return carry

        lax.fori_loop(0, c_chunks // _NBUF, body, 0)
        pltpu.sync_copy(s_v, out_hbm.at[wid])

    return k(weights, embs, idx3)


def _tc_loss_sum(scores, pos_cols):
    def body(s_ref, o_ref):
        s = s_ref[...]
        col = lax.broadcasted_iota(jnp.int32, s.shape, 1)
        z = jnp.where(col < pos_cols, s, -s)
        o_ref[...] = jnp.sum(jnp.log(jax.nn.sigmoid(z))).reshape(1, 1)

    out = pl.pallas_call(
        body,
        out_shape=jax.ShapeDtypeStruct((1, 1), jnp.float32),
    )(scores)
    return out[0, 0]


def kernel(input, embs, label, weights):
    n, d = embs.shape
    num_nodes = weights.shape[0]
    negs_flat = _sample_negs(n, num_nodes)
    idx = jnp.concatenate([label.astype(jnp.int32), jnp.asarray(negs_flat)])
    bb = n // _NW
    idx3 = (idx.reshape(_NUM_SAMPLED + 1, _NW, bb // _CHUNK, _CHUNK)
            .transpose(1, 0, 2, 3)
            .reshape(_NW, -1, _CHUNK))
    scores = _sc_scores(weights, embs, idx3)
    total = _tc_loss_sum(scores, bb)
    return -total / n


# speedup vs baseline: 2.3783x; 1.0580x over previous
import functools
import math

import jax
import jax.numpy as jnp
from jax import lax
from jax.experimental import pallas as pl
from jax.experimental.pallas import tpu as pltpu
from jax.experimental.pallas import tpu_sc as plsc

_NUM_SAMPLED = 5
_NW = 32
_CHUNK = 128
_NBUF = 4


@functools.lru_cache(maxsize=4)
def _sample_negs(n, num_nodes):
    with jax.ensure_compile_time_eval():
        kk = jnp.arange(num_nodes, dtype=jnp.float32)
        sw = (jnp.log(kk + 2.0) - jnp.log(kk + 1.0)) / math.log(num_nodes + 1)
        sw = sw / jnp.linalg.norm(sw)
        negs = jax.random.choice(jax.random.key(12345), num_nodes,
                                 shape=(n, _NUM_SAMPLED), replace=True, p=sw)
        return jax.device_get(negs.T.reshape(-1).astype(jnp.int32))


def _sc_scores(weights, eview, idx3, n, d):
    nw, c_chunks, chunk = idx3.shape
    bb = n // nw
    sub = bb // chunk
    groups = chunk // 16
    fts = d // 8
    erows_bt = n // chunk
    mesh = plsc.VectorSubcoreMesh(core_axis_name="c", subcore_axis_name="s")

    @functools.partial(
        pl.kernel,
        mesh=mesh,
        out_type=jax.ShapeDtypeStruct((nw, c_chunks * chunk), jnp.float32),
        compiler_params=pltpu.CompilerParams(use_tc_tiling_on_sc=False,
                                             needs_layout_passes=False),
        scratch_types=[
            pltpu.VMEM((c_chunks, chunk), jnp.int32),
            pltpu.VMEM((fts * sub * 8, chunk), jnp.float32),
            pltpu.VMEM((c_chunks * chunk,), jnp.float32),
        ] + [pltpu.VMEM((chunk, d), jnp.float32)] * _NBUF
          + [pltpu.SemaphoreType.DMA] * _NBUF,
    )
    def k(w_hbm, e_hbm, idx_hbm, out_hbm, idx_v, e_v, s_v, *bufsem):
        bufs = bufsem[:_NBUF]
        sems = bufsem[_NBUF:]
        wid = lax.axis_index("s") * 2 + lax.axis_index("c")
        pltpu.sync_copy(idx_hbm.at[wid], idx_v)
        for ft in range(fts):
            pltpu.sync_copy(
                e_hbm.at[pl.ds(ft * erows_bt * 8 + wid * sub * 8, sub * 8)],
                e_v.at[pl.ds(ft * sub * 8, sub * 8)])
        lanes = lax.iota(jnp.int32, 16)

        def compute(c, buf):
            j = lax.rem(c, sub)
            for g in range(groups):
                wrow = g * 16 + lanes
                acc = jnp.zeros((16,), jnp.float32)
                for dd in range(d):
                    col = jnp.full((16,), dd, jnp.int32)
                    wv = plsc.load_gather(buf, [wrow, col])
                    erow = (dd // 8) * sub * 8 + j * 8 + (dd % 8)
                    ev = e_v[erow, pl.ds(g * 16, 16)]
                    acc = acc + wv * ev
                s_v[pl.ds(c * chunk + g * 16, 16)] = acc

        def fire(c, k_):
            pltpu.async_copy(w_hbm.at[idx_v.at[c]], bufs[k_], sems[k_])

        def wait(k_):
            pltpu.make_async_copy(w_hbm.at[idx_v.at[0]], bufs[k_],
                                  sems[k_]).wait()

        for k_ in range(_NBUF):
            fire(k_, k_)

        def body(p, carry):
            for k_ in range(_NBUF):
                c = _NBUF * p + k_
                wait(k_)
                compute(c, bufs[k_])

                @pl.when(c + _NBUF < c_chunks)
                def _():
                    fire(c + _NBUF, k_)
            return carry

        lax.fori_loop(0, c_chunks // _NBUF, body, 0)
        pltpu.sync_copy(s_v, out_hbm.at[wid])

    return k(weights, eview, idx3)


def _tc_loss_sum(scores, pos_cols):
    def body(s_ref, o_ref):
        s = s_ref[...]
        col = lax.broadcasted_iota(jnp.int32, s.shape, 1)
        z = jnp.where(col < pos_cols, s, -s)
        o_ref[...] = jnp.sum(jnp.log(jax.nn.sigmoid(z))).reshape(1, 1)

    out = pl.pallas_call(
        body,
        out_shape=jax.ShapeDtypeStruct((1, 1), jnp.float32),
    )(scores)
    return out[0, 0]


def kernel(input, embs, label, weights):
    n, d = embs.shape
    num_nodes = weights.shape[0]
    negs_flat = _sample_negs(n, num_nodes)
    idx = jnp.concatenate([label.astype(jnp.int32), jnp.asarray(negs_flat)])
    bb = n // _NW
    idx3 = (idx.reshape(_NUM_SAMPLED + 1, _NW, bb // _CHUNK, _CHUNK)
            .transpose(1, 0, 2, 3)
            .reshape(_NW, -1, _CHUNK))
    eview = (embs.reshape(n // 128, 128, d // 8, 8)
             .transpose(2, 0, 3, 1)
             .reshape(n * d // 128, 128))
    scores = _sc_scores(weights, eview, idx3, n, d)
    total = _tc_loss_sum(scores, bb)
    return -total / n
